# baseline (speedup 1.0000x reference)
import functools

import jax
import jax.numpy as jnp
from jax import lax
from jax.experimental import pallas as pl
from jax.experimental.pallas import tpu as pltpu
from jax.experimental.pallas import tpu_sc as plsc

K = 64
CHUNK = 1024
BB = 256
BIGI = 2**30


def _topk_body(q_ref, wq_ref, wk_ref, mem_ref, outv_ref, outi_ref,
               a_s, rv_s, ri_s, s_s, *, cap, nchunk):
    i = pl.program_id(1)
    b = q_ref.shape[0]

    @pl.when(i == 0)
    def _init():
        qw = lax.dot_general(q_ref[...], wq_ref[...], (((1,), (1,)), ((), ())))
        a_s[...] = lax.dot_general(qw, wk_ref[...], (((1,), (0,)), ((), ())))
        rv_s[...] = jnp.full((b, K), -jnp.inf, jnp.float32)
        ri_s[...] = jnp.zeros((b, K), jnp.int32)

    s = lax.dot_general(a_s[...], mem_ref[...], (((1,), (1,)), ((), ())))
    cols = i * CHUNK + lax.broadcasted_iota(jnp.int32, (b, CHUNK), 1)
    s = jnp.where(cols < cap, s, -jnp.inf)
    s_s[...] = s

    tmin0 = jnp.min(rv_s[...], axis=1, keepdims=True)
    cnt = jnp.sum((s > tmin0).astype(jnp.int32), axis=1)
    nrounds = jnp.minimum(jnp.max(cnt), K)

    kio = lax.broadcasted_iota(jnp.int32, (b, K), 1)

    def _round(_, carry):
        sv = s_s[...]
        rv = rv_s[...]
        ri = ri_s[...]
        tmin = jnp.min(rv, axis=1, keepdims=True)
        m = jnp.max(sv, axis=1, keepdims=True)
        ins = m > tmin
        p = jnp.min(jnp.where(sv == m, cols, BIGI), axis=1, keepdims=True)
        s_s[...] = jnp.where(cols == p, -jnp.inf, sv)
        q = jnp.min(jnp.where(rv == tmin, kio, BIGI), axis=1, keepdims=True)
        sel = (kio == q) & ins
        rv_s[...] = jnp.where(sel, m, rv)
        ri_s[...] = jnp.where(sel, p, ri)
        return carry

    lax.fori_loop(0, nrounds, _round, 0)

    @pl.when(i == nchunk - 1)
    def _fin():
        outv_ref[...] = rv_s[...]
        outi_ref[...] = ri_s[...]


def _read_body(rv_ref, tk_ref, rows_ref, wv_ref, out_ref):
    rv = rv_ref[...]
    tk = tk_ref[0, 0]
    rank = jnp.sum((rv[:, None, :] > rv[:, :, None]).astype(jnp.float32),
                   axis=2)
    m = jnp.max(rv, axis=1, keepdims=True)
    e = jnp.exp(rv - m)
    e = jnp.where(rank < tk.astype(jnp.float32), e, 0.0)
    w = e / jnp.sum(e, axis=1, keepdims=True)
    rows = rows_ref[...]
    u = jnp.sum(w[:, :, None] * rows, axis=1)[:, :wv_ref.shape[1]]
    out_ref[...] = lax.dot_general(u, wv_ref[...], (((1,), (1,)), ((), ())))


def _sc_gather(idx_flat, table):
    n = idx_flat.shape[0]
    d = table.shape[1]
    info = plsc.get_sparse_core_info()
    nw = info.num_cores * info.num_subcores
    gchunk = 128
    per_w = n // nw
    niter = per_w // gchunk
    mesh = plsc.VectorSubcoreMesh(core_axis_name="c", subcore_axis_name="s")

    @functools.partial(
        pl.kernel, mesh=mesh,
        out_type=jax.ShapeDtypeStruct((n, d), jnp.float32),
        scratch_types=[
            pltpu.VMEM((gchunk,), jnp.int32),
            pltpu.VMEM((gchunk, d), jnp.float32),
            pltpu.SemaphoreType.DMA,
        ],
    )
    def gather_k(idx_hbm, table_hbm, out_hbm, idx_v, rows_v, sem):
        wid = lax.axis_index("s") * info.num_cores + lax.axis_index("c")
        base = wid * per_w
        for j in range(niter):
            off = base + j * gchunk
            pltpu.sync_copy(idx_hbm.at[pl.ds(off, gchunk)], idx_v)
            pltpu.async_copy(table_hbm.at[idx_v], rows_v, sem).wait()
            pltpu.sync_copy(rows_v, out_hbm.at[pl.ds(off, gchunk)])

    return gather_k(idx_flat, table)


def kernel(query, top_k, memory, Wq, Wk, Wv):
    b, d = query.shape
    cap = memory.shape[0]
    nchunk = (cap + CHUNK - 1) // CHUNK
    dp = 128
    memp = jnp.pad(memory, ((0, 0), (0, dp - d)))

    vals, idx = pl.pallas_call(
        functools.partial(_topk_body, cap=cap, nchunk=nchunk),
        grid=(b // BB, nchunk),
        in_specs=[
            pl.BlockSpec((BB, d), lambda ib, ic: (ib, 0)),
            pl.BlockSpec((d, d), lambda ib, ic: (0, 0)),
            pl.BlockSpec((d, d), lambda ib, ic: (0, 0)),
            pl.BlockSpec((CHUNK, d), lambda ib, ic: (ic, 0)),
        ],
        out_specs=[
            pl.BlockSpec((BB, K), lambda ib, ic: (ib, 0)),
            pl.BlockSpec((BB, K), lambda ib, ic: (ib, 0)),
        ],
        out_shape=[
            jax.ShapeDtypeStruct((b, K), jnp.float32),
            jax.ShapeDtypeStruct((b, K), jnp.int32),
        ],
        scratch_shapes=[
            pltpu.VMEM((BB, d), jnp.float32),
            pltpu.VMEM((BB, K), jnp.float32),
            pltpu.VMEM((BB, K), jnp.int32),
            pltpu.VMEM((BB, CHUNK), jnp.float32),
        ],
    )(query, Wq, Wk, memory)

    rows = _sc_gather(idx.reshape(b * K), memp)
    rows3 = rows.reshape(b, K, dp)
    tk = jnp.asarray(top_k, jnp.int32).reshape(1, 1)

    bb = 128
    out = pl.pallas_call(
        _read_body,
        grid=(b // bb,),
        in_specs=[
            pl.BlockSpec((bb, K), lambda i: (i, 0)),
            pl.BlockSpec((1, 1), lambda i: (0, 0), memory_space=pltpu.SMEM),
            pl.BlockSpec((bb, K, dp), lambda i: (i, 0, 0)),
            pl.BlockSpec((d, d), lambda i: (0, 0)),
        ],
        out_specs=pl.BlockSpec((bb, d), lambda i: (i, 0)),
        out_shape=jax.ShapeDtypeStruct((b, d), jnp.float32),
    )(vals, tk, rows3, Wv)
    return out


# speedup vs baseline: 1.0787x; 1.0787x over previous
import functools

import jax
import jax.numpy as jnp
from jax import lax
from jax.experimental import pallas as pl
from jax.experimental.pallas import tpu as pltpu
from jax.experimental.pallas import tpu_sc as plsc

K = 64
CHUNK = 1024
BIGI = 2**30


def _topk_body(q_ref, wq_ref, wk_ref, mem_ref, outv_ref, outi_ref,
               a_s, rv_s, ri_s, s_s, *, cap, nchunk):
    i = pl.program_id(0)
    b = q_ref.shape[0]

    @pl.when(i == 0)
    def _init():
        qw = lax.dot_general(q_ref[...], wq_ref[...], (((1,), (1,)), ((), ())))
        a_s[...] = lax.dot_general(qw, wk_ref[...], (((1,), (0,)), ((), ())))
        rv_s[...] = jnp.full((b, K), -jnp.inf, jnp.float32)
        ri_s[...] = jnp.zeros((b, K), jnp.int32)

    s = lax.dot_general(a_s[...], mem_ref[...], (((1,), (1,)), ((), ())))
    cols = i * CHUNK + lax.broadcasted_iota(jnp.int32, (b, CHUNK), 1)
    s = jnp.where(cols < cap, s, -jnp.inf)
    s_s[...] = s

    tmin0 = jnp.min(rv_s[...], axis=1, keepdims=True)
    cnt = jnp.sum((s > tmin0).astype(jnp.int32), axis=1)
    nrounds = jnp.minimum(jnp.max(cnt), K)

    kio = lax.broadcasted_iota(jnp.int32, (b, K), 1)

    def _round(_, carry):
        sv = s_s[...]
        rv = rv_s[...]
        ri = ri_s[...]
        tmin = jnp.min(rv, axis=1, keepdims=True)
        m = jnp.max(sv, axis=1, keepdims=True)
        ins = m > tmin
        p = jnp.min(jnp.where(sv == m, cols, BIGI), axis=1, keepdims=True)
        s_s[...] = jnp.where(cols == p, -jnp.inf, sv)
        q = jnp.min(jnp.where(rv == tmin, kio, BIGI), axis=1, keepdims=True)
        sel = (kio == q) & ins
        rv_s[...] = jnp.where(sel, m, rv)
        ri_s[...] = jnp.where(sel, p, ri)
        return carry

    lax.fori_loop(0, nrounds, _round, 0)

    @pl.when(i == nchunk - 1)
    def _fin():
        outv_ref[...] = rv_s[...]
        outi_ref[...] = ri_s[...]


def _read_body(rv_ref, tk_ref, rows_ref, wv_ref, out_ref):
    rv = rv_ref[...]
    tk = tk_ref[0, 0]
    rank = jnp.sum((rv[:, None, :] > rv[:, :, None]).astype(jnp.float32),
                   axis=2)
    m = jnp.max(rv, axis=1, keepdims=True)
    e = jnp.exp(rv - m)
    e = jnp.where(rank < tk.astype(jnp.float32), e, 0.0)
    w = e / jnp.sum(e, axis=1, keepdims=True)
    rows = rows_ref[...]
    u = jnp.sum(w[:, :, None] * rows, axis=1)[:, :wv_ref.shape[1]]
    out_ref[...] = lax.dot_general(u, wv_ref[...], (((1,), (1,)), ((), ())))


def _sc_gather(idx_flat, table):
    n = idx_flat.shape[0]
    d = table.shape[1]
    info = plsc.get_sparse_core_info()
    nw = info.num_cores * info.num_subcores
    gchunk = 128
    per_w = n // nw
    niter = per_w // gchunk
    mesh = plsc.VectorSubcoreMesh(core_axis_name="c", subcore_axis_name="s")

    @functools.partial(
        pl.kernel, mesh=mesh,
        out_type=jax.ShapeDtypeStruct((n, d), jnp.float32),
        scratch_types=[
            pltpu.VMEM((gchunk,), jnp.int32),
            pltpu.VMEM((gchunk, d), jnp.float32),
            pltpu.SemaphoreType.DMA,
        ],
    )
    def gather_k(idx_hbm, table_hbm, out_hbm, idx_v, rows_v, sem):
        wid = lax.axis_index("s") * info.num_cores + lax.axis_index("c")
        base = wid * per_w
        for j in range(niter):
            off = base + j * gchunk
            pltpu.sync_copy(idx_hbm.at[pl.ds(off, gchunk)], idx_v)
            pltpu.async_copy(table_hbm.at[idx_v], rows_v, sem).wait()
            pltpu.sync_copy(rows_v, out_hbm.at[pl.ds(off, gchunk)])

    return gather_k(idx_flat, table)


def kernel(query, top_k, memory, Wq, Wk, Wv):
    b, d = query.shape
    cap = memory.shape[0]
    nchunk = (cap + CHUNK - 1) // CHUNK
    dp = 128
    memp = jnp.pad(memory, ((0, 0), (0, dp - d)))

    vals, idx = pl.pallas_call(
        functools.partial(_topk_body, cap=cap, nchunk=nchunk),
        grid=(nchunk,),
        in_specs=[
            pl.BlockSpec((b, d), lambda i: (0, 0)),
            pl.BlockSpec((d, d), lambda i: (0, 0)),
            pl.BlockSpec((d, d), lambda i: (0, 0)),
            pl.BlockSpec((CHUNK, d), lambda i: (i, 0)),
        ],
        out_specs=[
            pl.BlockSpec((b, K), lambda i: (0, 0)),
            pl.BlockSpec((b, K), lambda i: (0, 0)),
        ],
        out_shape=[
            jax.ShapeDtypeStruct((b, K), jnp.float32),
            jax.ShapeDtypeStruct((b, K), jnp.int32),
        ],
        scratch_shapes=[
            pltpu.VMEM((b, d), jnp.float32),
            pltpu.VMEM((b, K), jnp.float32),
            pltpu.VMEM((b, K), jnp.int32),
            pltpu.VMEM((b, CHUNK), jnp.float32),
        ],
    )(query, Wq, Wk, memory)

    rows = _sc_gather(idx.reshape(b * K), memp)
    rows3 = rows.reshape(b, K, dp)
    tk = jnp.asarray(top_k, jnp.int32).reshape(1, 1)

    bb = 128
    out = pl.pallas_call(
        _read_body,
        grid=(b // bb,),
        in_specs=[
            pl.BlockSpec((bb, K), lambda i: (i, 0)),
            pl.BlockSpec((1, 1), lambda i: (0, 0), memory_space=pltpu.SMEM),
            pl.BlockSpec((bb, K, dp), lambda i: (i, 0, 0)),
            pl.BlockSpec((d, d), lambda i: (0, 0)),
        ],
        out_specs=pl.BlockSpec((bb, d), lambda i: (i, 0)),
        out_shape=jax.ShapeDtypeStruct((b, d), jnp.float32),
    )(vals, tk, rows3, Wv)
    return out
